# baseline (speedup 1.0000x reference)
import jax
import jax.numpy as jnp
from jax.experimental import pallas as pl
from jax.experimental.pallas import tpu as pltpu


def _add_kernel(x_ref, p_ref, o_ref):
    o_ref[0, 0] = x_ref[0, 0] + p_ref[0]


def kernel(x, P):
    B, W, D = x.shape
    half = W // 2
    x4 = x.reshape(B, 2, half, D)
    P3 = P.reshape(2, half, D)
    out = pl.pallas_call(
        _add_kernel,
        grid=(2 * B,),
        in_specs=[
            pl.BlockSpec((1, 1, half, D), lambda k: (k % B, k // B, 0, 0)),
            pl.BlockSpec((1, half, D), lambda k: (k // B, 0, 0)),
        ],
        out_specs=pl.BlockSpec((1, 1, half, D), lambda k: (k % B, k // B, 0, 0)),
        out_shape=jax.ShapeDtypeStruct((B, 2, half, D), x.dtype),
        compiler_params=pltpu.CompilerParams(
            dimension_semantics=("arbitrary",),
        ),
    )(x4, P3)
    return out.reshape(B, W, D)
